# baseline (speedup 1.0000x reference)
import functools

import jax
import jax.numpy as jnp
from jax import lax
from jax.experimental import pallas as pl
from jax.experimental.pallas import tpu as pltpu
from jax.experimental.pallas import tpu_sc as plsc

VOCAB = 100000
DIM = 128
HID = 16
CLASSES = 2
B = 4096
L = 200

NC = 2
NS = 16
NW = NC * NS
ROWS_PER_W = B // NW

PACK_ROWS = 12544
VOCAB_PAD = 8 * PACK_ROWS
_GRID = 4
_BROWS = PACK_ROWS // _GRID

CHUNK_A = 128
CHUNK_B = 72


def _proj_body(*refs):
    t_refs = refs[:8]
    w1_ref, b1_ref, out_ref = refs[8:]
    w1 = w1_ref[...]
    b1 = b1_ref[...]
    parts = [
        jnp.dot(t[...], w1, preferred_element_type=jnp.float32) + b1
        for t in t_refs
    ]
    out_ref[...] = jnp.concatenate(parts, axis=1)


def _project(table, w1, b1row):
    in_specs = [
        pl.BlockSpec((_BROWS, DIM), (lambda i, s=s: (s * _GRID + i, 0)))
        for s in range(8)
    ] + [
        pl.BlockSpec((DIM, HID), lambda i: (0, 0)),
        pl.BlockSpec((1, HID), lambda i: (0, 0)),
    ]
    return pl.pallas_call(
        _proj_body,
        grid=(_GRID,),
        in_specs=in_specs,
        out_specs=pl.BlockSpec((_BROWS, 8 * HID), lambda i: (i, 0)),
        out_shape=jax.ShapeDtypeStruct((PACK_ROWS, 8 * HID), jnp.float32),
    )(*([table] * 8), w1, b1row)


def _accum(buf):
    a0 = buf[0]
    a1 = buf[1]
    a2 = buf[2]
    a3 = buf[3]
    for j in range(4, L, 4):
        a0 = a0 + buf[j]
        a1 = a1 + buf[j + 1]
        a2 = a2 + buf[j + 2]
        a3 = a3 + buf[j + 3]
    return (a0 + a1) + (a2 + a3)


def _gather_sum(xm, p):
    mesh = plsc.VectorSubcoreMesh(core_axis_name="c", subcore_axis_name="s")
    NBUF = 4
    STEP = 8

    @functools.partial(
        pl.kernel,
        out_type=jax.ShapeDtypeStruct((B // 8, 8 * HID), jnp.float32),
        mesh=mesh,
        scratch_types=[
            pltpu.VMEM((ROWS_PER_W, L), jnp.int32),
            pltpu.VMEM((NBUF, L, HID), jnp.float32),
            pltpu.VMEM((ROWS_PER_W // 8, 8 * HID), jnp.float32),
            pltpu.SemaphoreType.DMA,
        ] + [pltpu.SemaphoreType.DMA] * NBUF,
        compiler_params=pltpu.CompilerParams(use_tc_tiling_on_sc=False),
    )
    def body(x_hbm, p_hbm, out_hbm, idx_v, bufs, out_v, semi, *sems):
        wid = lax.axis_index("s") * NC + lax.axis_index("c")
        pltpu.async_copy(
            x_hbm.at[pl.ds(ROWS_PER_W * wid, ROWS_PER_W)], idx_v, semi).wait()

        def fire_row(r, k):
            pltpu.async_copy(p_hbm.at[idx_v.at[r, pl.ds(0, CHUNK_A)]],
                             bufs.at[k, pl.ds(0, CHUNK_A)], sems[k])
            pltpu.async_copy(p_hbm.at[idx_v.at[r, pl.ds(CHUNK_A, CHUNK_B)]],
                             bufs.at[k, pl.ds(CHUNK_A, CHUNK_B)], sems[k])

        def wait_row(k):
            pltpu.make_async_copy(p_hbm.at[pl.ds(0, L)],
                                  bufs.at[k], sems[k]).wait()

        for k in range(NBUF):
            fire_row(k, k)

        def step(i, _):
            for k in range(STEP):
                kb = k % NBUF
                wait_row(kb)
                acc = _accum(bufs.at[kb])
                out_v[i, pl.ds(HID * k, HID)] = acc

                @pl.when(STEP * i + k + NBUF < ROWS_PER_W)
                def _():
                    fire_row(STEP * i + k + NBUF, kb)

            return 0

        lax.fori_loop(0, ROWS_PER_W // STEP, step, 0)
        pltpu.sync_copy(
            out_v,
            out_hbm.at[pl.ds(wid * (ROWS_PER_W // 8), ROWS_PER_W // 8)])

    return body(xm, p)


def _head_body(s8_ref, yf_ref, m_ref, b2_ref, out_ref):
    h = jnp.maximum(s8_ref[...] * (1.0 / L), 0.0)
    lg = jnp.dot(h, m_ref[...], preferred_element_type=jnp.float32)
    b2 = b2_ref[...]
    l0 = lg[:, :8] + b2[0, 0]
    l1 = lg[:, 8:] + b2[0, 1]
    mx = jnp.maximum(l0, l1)
    lse = mx + jnp.log(jnp.exp(l0 - mx) + jnp.exp(l1 - mx))
    yf = yf_ref[...]
    picked = l0 + yf * (l1 - l0)
    out_ref[...] = (jnp.sum(lse - picked) * (1.0 / B)).reshape(1, 1)


def _head(s8, yf, m, b2row):
    return pl.pallas_call(
        _head_body,
        out_shape=jax.ShapeDtypeStruct((1, 1), jnp.float32),
    )(s8, yf, m, b2row)


def kernel(x_, y_, table, W1, b1, W2, b2):
    p8 = _project(table, W1, b1.reshape(1, HID))
    p = p8.reshape(VOCAB_PAD, HID)

    xi = x_.astype(jnp.int32)
    xm = 8 * (xi % PACK_ROWS) + xi // PACK_ROWS

    s8 = _gather_sum(xm, p)

    eye8 = jnp.eye(8, dtype=jnp.float32)
    m = jnp.concatenate(
        [jnp.kron(eye8, W2[:, 0:1]), jnp.kron(eye8, W2[:, 1:2])], axis=1)
    yf = y_.astype(jnp.float32).reshape(B // 8, 8)

    out = _head(s8, yf, m, b2.reshape(1, CLASSES))
    return out[0, 0]


# speedup vs baseline: 1.0573x; 1.0573x over previous
import functools

import jax
import jax.numpy as jnp
from jax import lax
from jax.experimental import pallas as pl
from jax.experimental.pallas import tpu as pltpu
from jax.experimental.pallas import tpu_sc as plsc

VOCAB = 100000
DIM = 128
HID = 16
CLASSES = 2
B = 4096
L = 200

NC = 2
NS = 16
NW = NC * NS
ROWS_PER_W = B // NW

PACK_ROWS = 12544
VOCAB_PAD = 8 * PACK_ROWS
_GRID = 4
_BROWS = PACK_ROWS // _GRID

CHUNK_A = 104
CHUNK_B = 96


def _proj_body(*refs):
    t_refs = refs[:8]
    w1_ref, b1_ref, out_ref = refs[8:]
    w1 = w1_ref[...]
    b1 = b1_ref[...]
    parts = [
        jnp.dot(t[...], w1, preferred_element_type=jnp.float32) + b1
        for t in t_refs
    ]
    out_ref[...] = jnp.concatenate(parts, axis=1)


def _project(table, w1, b1row):
    in_specs = [
        pl.BlockSpec((_BROWS, DIM), (lambda i, s=s: (s * _GRID + i, 0)))
        for s in range(8)
    ] + [
        pl.BlockSpec((DIM, HID), lambda i: (0, 0)),
        pl.BlockSpec((1, HID), lambda i: (0, 0)),
    ]
    return pl.pallas_call(
        _proj_body,
        grid=(_GRID,),
        in_specs=in_specs,
        out_specs=pl.BlockSpec((_BROWS, 8 * HID), lambda i: (i, 0)),
        out_shape=jax.ShapeDtypeStruct((PACK_ROWS, 8 * HID), jnp.float32),
    )(*([table] * 8), w1, b1row)


def _accum(buf):
    a0 = buf[0]
    a1 = buf[1]
    a2 = buf[2]
    a3 = buf[3]
    for j in range(4, L, 4):
        a0 = a0 + buf[j]
        a1 = a1 + buf[j + 1]
        a2 = a2 + buf[j + 2]
        a3 = a3 + buf[j + 3]
    return (a0 + a1) + (a2 + a3)


def _gather_sum(x2, p):
    mesh = plsc.VectorSubcoreMesh(core_axis_name="c", subcore_axis_name="s")
    NBUF = 8
    STEP = 8

    @functools.partial(
        pl.kernel,
        out_type=jax.ShapeDtypeStruct((B // 8, 8 * HID), jnp.float32),
        mesh=mesh,
        scratch_types=[
            pltpu.VMEM((2 * ROWS_PER_W, 128), jnp.int32),
            pltpu.VMEM((NBUF, L, HID), jnp.float32),
            pltpu.VMEM((ROWS_PER_W // 8, 8 * HID), jnp.float32),
            pltpu.SemaphoreType.DMA,
        ] + [pltpu.SemaphoreType.DMA] * NBUF,
        compiler_params=pltpu.CompilerParams(use_tc_tiling_on_sc=False),
    )
    def body(x_hbm, p_hbm, out_hbm, idx_v, bufs, out_v, semi, *sems):
        wid = lax.axis_index("s") * NC + lax.axis_index("c")
        pltpu.async_copy(
            x_hbm.at[pl.ds(2 * ROWS_PER_W * wid, 2 * ROWS_PER_W)],
            idx_v, semi).wait()

        def fire_row(r, k):
            pltpu.async_copy(p_hbm.at[idx_v.at[2 * r, pl.ds(0, CHUNK_A)]],
                             bufs.at[k, pl.ds(0, CHUNK_A)], sems[k])
            pltpu.async_copy(p_hbm.at[idx_v.at[2 * r + 1, pl.ds(0, CHUNK_B)]],
                             bufs.at[k, pl.ds(CHUNK_A, CHUNK_B)], sems[k])

        def wait_row(k):
            pltpu.make_async_copy(p_hbm.at[pl.ds(0, L)],
                                  bufs.at[k], sems[k]).wait()

        for k in range(NBUF):
            fire_row(k, k)

        def step(i, _):
            for k in range(STEP):
                kb = k % NBUF
                wait_row(kb)
                acc = _accum(bufs.at[kb])
                out_v[i, pl.ds(HID * k, HID)] = acc

                @pl.when(STEP * i + k + NBUF < ROWS_PER_W)
                def _():
                    fire_row(STEP * i + k + NBUF, kb)

            return 0

        lax.fori_loop(0, ROWS_PER_W // STEP, step, 0)
        pltpu.sync_copy(
            out_v,
            out_hbm.at[pl.ds(wid * (ROWS_PER_W // 8), ROWS_PER_W // 8)])

    return body(x2, p)


def _head_body(s8_ref, yf_ref, m_ref, b2_ref, out_ref):
    h = jnp.maximum(s8_ref[...] * (1.0 / L), 0.0)
    lg = jnp.dot(h, m_ref[...], preferred_element_type=jnp.float32)
    b2 = b2_ref[...]
    l0 = lg[:, :8] + b2[0, 0]
    l1 = lg[:, 8:] + b2[0, 1]
    mx = jnp.maximum(l0, l1)
    lse = mx + jnp.log(jnp.exp(l0 - mx) + jnp.exp(l1 - mx))
    yf = yf_ref[...]
    picked = l0 + yf * (l1 - l0)
    out_ref[...] = (jnp.sum(lse - picked) * (1.0 / B)).reshape(1, 1)


def _head(s8, yf, m, b2row):
    return pl.pallas_call(
        _head_body,
        out_shape=jax.ShapeDtypeStruct((1, 1), jnp.float32),
    )(s8, yf, m, b2row)


def kernel(x_, y_, table, W1, b1, W2, b2):
    p8 = _project(table, W1, b1.reshape(1, HID))
    p = p8.reshape(VOCAB_PAD, HID)

    xi = x_.astype(jnp.int32)
    xm = 8 * (xi % PACK_ROWS) + xi // PACK_ROWS
    za = jnp.zeros((B, 128 - CHUNK_A), jnp.int32)
    zb = jnp.zeros((B, 128 - CHUNK_B), jnp.int32)
    x2 = jnp.concatenate(
        [xm[:, :CHUNK_A], za, xm[:, CHUNK_A:], zb],
        axis=1).reshape(2 * B, 128)

    s8 = _gather_sum(x2, p)

    eye8 = jnp.eye(8, dtype=jnp.float32)
    m = jnp.concatenate(
        [jnp.kron(eye8, W2[:, 0:1]), jnp.kron(eye8, W2[:, 1:2])], axis=1)
    yf = y_.astype(jnp.float32).reshape(B // 8, 8)

    out = _head(s8, yf, m, b2.reshape(1, CLASSES))
    return out[0, 0]


# speedup vs baseline: 1.1143x; 1.0539x over previous
import functools

import jax
import jax.numpy as jnp
from jax import lax
from jax.experimental import pallas as pl
from jax.experimental.pallas import tpu as pltpu
from jax.experimental.pallas import tpu_sc as plsc

VOCAB = 100000
DIM = 128
HID = 16
CLASSES = 2
B = 4096
L = 200

NC = 2
NS = 16
NW = NC * NS
ROWS_PER_W = B // NW

PACK_ROWS = 12544
VOCAB_PAD = 8 * PACK_ROWS
_GRID = 4
_BROWS = PACK_ROWS // _GRID

CHUNK_A = 104
CHUNK_B = 96


def _proj_body(*refs):
    t_refs = refs[:8]
    x_ref, w1_ref, b1_ref, out_ref, outa_ref, outb_ref = refs[8:]
    w1 = w1_ref[...]
    b1 = b1_ref[...]
    parts = [
        jnp.dot(t[...], w1, preferred_element_type=jnp.float32) + b1
        for t in t_refs
    ]
    out_ref[...] = jnp.concatenate(parts, axis=1)
    xi = x_ref[...]
    xm = 8 * (xi % PACK_ROWS) + xi // PACK_ROWS
    xrows = xi.shape[0]
    za = jnp.zeros((xrows, 128 - CHUNK_A), jnp.int32)
    zb = jnp.zeros((xrows, 128 - CHUNK_B), jnp.int32)
    outa_ref[...] = jnp.concatenate([xm[:, :CHUNK_A], za], axis=1)
    outb_ref[...] = jnp.concatenate([xm[:, CHUNK_A:], zb], axis=1)


def _project(table, w1, b1row, x):
    xrows = B // _GRID
    in_specs = [
        pl.BlockSpec((_BROWS, DIM), (lambda i, s=s: (s * _GRID + i, 0)))
        for s in range(8)
    ] + [
        pl.BlockSpec((xrows, L), lambda i: (i, 0)),
        pl.BlockSpec((DIM, HID), lambda i: (0, 0)),
        pl.BlockSpec((1, HID), lambda i: (0, 0)),
    ]
    return pl.pallas_call(
        _proj_body,
        grid=(_GRID,),
        in_specs=in_specs,
        out_specs=[
            pl.BlockSpec((_BROWS, 8 * HID), lambda i: (i, 0)),
            pl.BlockSpec((xrows, 128), lambda i: (i, 0)),
            pl.BlockSpec((xrows, 128), lambda i: (i, 0)),
        ],
        out_shape=[
            jax.ShapeDtypeStruct((PACK_ROWS, 8 * HID), jnp.float32),
            jax.ShapeDtypeStruct((B, 128), jnp.int32),
            jax.ShapeDtypeStruct((B, 128), jnp.int32),
        ],
    )(*([table] * 8), x, w1, b1row)


def _accum(buf):
    a0 = buf[0]
    a1 = buf[1]
    a2 = buf[2]
    a3 = buf[3]
    for j in range(4, L, 4):
        a0 = a0 + buf[j]
        a1 = a1 + buf[j + 1]
        a2 = a2 + buf[j + 2]
        a3 = a3 + buf[j + 3]
    return (a0 + a1) + (a2 + a3)


def _gather_sum(xa, xb, p):
    mesh = plsc.VectorSubcoreMesh(core_axis_name="c", subcore_axis_name="s")
    NBUF = 8
    STEP = 8

    @functools.partial(
        pl.kernel,
        out_type=jax.ShapeDtypeStruct((B // 8, 8 * HID), jnp.float32),
        mesh=mesh,
        scratch_types=[
            pltpu.VMEM((ROWS_PER_W, 128), jnp.int32),
            pltpu.VMEM((ROWS_PER_W, 128), jnp.int32),
            pltpu.VMEM((NBUF, L, HID), jnp.float32),
            pltpu.VMEM((ROWS_PER_W // 8, 8 * HID), jnp.float32),
            pltpu.SemaphoreType.DMA,
        ] + [pltpu.SemaphoreType.DMA] * NBUF,
        compiler_params=pltpu.CompilerParams(use_tc_tiling_on_sc=False),
    )
    def body(xa_hbm, xb_hbm, p_hbm, out_hbm, idx_a, idx_b, bufs, out_v,
             semi, *sems):
        wid = lax.axis_index("s") * NC + lax.axis_index("c")
        pltpu.async_copy(
            xa_hbm.at[pl.ds(ROWS_PER_W * wid, ROWS_PER_W)], idx_a, semi)
        pltpu.async_copy(
            xb_hbm.at[pl.ds(ROWS_PER_W * wid, ROWS_PER_W)], idx_b, semi).wait()
        pltpu.make_async_copy(
            xa_hbm.at[pl.ds(0, ROWS_PER_W)], idx_a, semi).wait()

        def fire_row(r, k):
            pltpu.async_copy(p_hbm.at[idx_a.at[r, pl.ds(0, CHUNK_A)]],
                             bufs.at[k, pl.ds(0, CHUNK_A)], sems[k])
            pltpu.async_copy(p_hbm.at[idx_b.at[r, pl.ds(0, CHUNK_B)]],
                             bufs.at[k, pl.ds(CHUNK_A, CHUNK_B)], sems[k])

        def wait_row(k):
            pltpu.make_async_copy(p_hbm.at[pl.ds(0, L)],
                                  bufs.at[k], sems[k]).wait()

        for k in range(NBUF):
            fire_row(k, k)

        def step(i, _):
            for k in range(STEP):
                kb = k % NBUF
                wait_row(kb)
                acc = _accum(bufs.at[kb])
                out_v[i, pl.ds(HID * k, HID)] = acc

                @pl.when(STEP * i + k + NBUF < ROWS_PER_W)
                def _():
                    fire_row(STEP * i + k + NBUF, kb)

            return 0

        lax.fori_loop(0, ROWS_PER_W // STEP, step, 0)
        pltpu.sync_copy(
            out_v,
            out_hbm.at[pl.ds(wid * (ROWS_PER_W // 8), ROWS_PER_W // 8)])

    return body(xa, xb, p)


def _head_body(s8_ref, yf_ref, m_ref, b2_ref, out_ref):
    h = jnp.maximum(s8_ref[...] * (1.0 / L), 0.0)
    lg = jnp.dot(h, m_ref[...], preferred_element_type=jnp.float32)
    b2 = b2_ref[...]
    l0 = lg[:, :8] + b2[0, 0]
    l1 = lg[:, 8:] + b2[0, 1]
    mx = jnp.maximum(l0, l1)
    lse = mx + jnp.log(jnp.exp(l0 - mx) + jnp.exp(l1 - mx))
    yf = yf_ref[...]
    picked = l0 + yf * (l1 - l0)
    out_ref[...] = (jnp.sum(lse - picked) * (1.0 / B)).reshape(1, 1)


def _head(s8, yf, m, b2row):
    return pl.pallas_call(
        _head_body,
        out_shape=jax.ShapeDtypeStruct((1, 1), jnp.float32),
    )(s8, yf, m, b2row)


def kernel(x_, y_, table, W1, b1, W2, b2):
    p8, xa, xb = _project(table, W1, b1.reshape(1, HID), x_.astype(jnp.int32))
    p = p8.reshape(VOCAB_PAD, HID)

    s8 = _gather_sum(xa, xb, p)

    eye8 = jnp.eye(8, dtype=jnp.float32)
    m = jnp.concatenate(
        [jnp.kron(eye8, W2[:, 0:1]), jnp.kron(eye8, W2[:, 1:2])], axis=1)
    yf = y_.astype(jnp.float32).reshape(B // 8, 8)

    out = _head(s8, yf, m, b2.reshape(1, CLASSES))
    return out[0, 0]


# speedup vs baseline: 1.1224x; 1.0073x over previous
import functools

import jax
import jax.numpy as jnp
from jax import lax
from jax.experimental import pallas as pl
from jax.experimental.pallas import tpu as pltpu
from jax.experimental.pallas import tpu_sc as plsc

VOCAB = 100000
DIM = 128
HID = 16
CLASSES = 2
B = 4096
L = 200

NC = 2
NS = 16
NW = NC * NS
ROWS_PER_W = B // NW

PACK_ROWS = 12544
VOCAB_PAD = 8 * PACK_ROWS
_GRID = 4
_BROWS = PACK_ROWS // _GRID

CHUNK_A = 104
CHUNK_B = 96


def _proj_body(*refs):
    t_refs = refs[:8]
    x_ref, w1_ref, b1_ref, out_ref, outa_ref, outb_ref = refs[8:]
    w1 = w1_ref[...]
    b1 = b1_ref[...]
    parts = [
        jnp.dot(t[...], w1, preferred_element_type=jnp.float32) + b1
        for t in t_refs
    ]
    out_ref[...] = jnp.concatenate(parts, axis=1)
    xi = x_ref[...]
    xm = 8 * (xi % PACK_ROWS) + xi // PACK_ROWS
    xrows = xi.shape[0]
    za = jnp.zeros((xrows, 128 - CHUNK_A), jnp.int32)
    zb = jnp.zeros((xrows, 128 - CHUNK_B), jnp.int32)
    outa_ref[...] = jnp.concatenate([xm[:, :CHUNK_A], za], axis=1)
    outb_ref[...] = jnp.concatenate([xm[:, CHUNK_A:], zb], axis=1)


def _project(table, w1, b1row, x):
    xrows = B // _GRID
    in_specs = [
        pl.BlockSpec((_BROWS, DIM), (lambda i, s=s: (s * _GRID + i, 0)))
        for s in range(8)
    ] + [
        pl.BlockSpec((xrows, L), lambda i: (i, 0)),
        pl.BlockSpec((DIM, HID), lambda i: (0, 0)),
        pl.BlockSpec((1, HID), lambda i: (0, 0)),
    ]
    return pl.pallas_call(
        _proj_body,
        grid=(_GRID,),
        in_specs=in_specs,
        out_specs=[
            pl.BlockSpec((_BROWS, 8 * HID), lambda i: (i, 0)),
            pl.BlockSpec((xrows, 128), lambda i: (i, 0)),
            pl.BlockSpec((xrows, 128), lambda i: (i, 0)),
        ],
        out_shape=[
            jax.ShapeDtypeStruct((PACK_ROWS, 8 * HID), jnp.float32),
            jax.ShapeDtypeStruct((B, 128), jnp.int32),
            jax.ShapeDtypeStruct((B, 128), jnp.int32),
        ],
    )(*([table] * 8), x, w1, b1row)


def _accum(buf):
    a0 = buf[0]
    a1 = buf[1]
    a2 = buf[2]
    a3 = buf[3]
    for j in range(4, L, 4):
        a0 = a0 + buf[j]
        a1 = a1 + buf[j + 1]
        a2 = a2 + buf[j + 2]
        a3 = a3 + buf[j + 3]
    return (a0 + a1) + (a2 + a3)


def _gather_sum(xa, xb, p):
    mesh = plsc.VectorSubcoreMesh(core_axis_name="c", subcore_axis_name="s")
    NBUF = 8
    STEP = 8

    @functools.partial(
        pl.kernel,
        out_type=jax.ShapeDtypeStruct((B // 8, 8 * HID), jnp.float32),
        mesh=mesh,
        scratch_types=[
            pltpu.VMEM((ROWS_PER_W, CHUNK_A), jnp.int32),
            pltpu.VMEM((ROWS_PER_W, CHUNK_B), jnp.int32),
            pltpu.VMEM((NBUF, L, HID), jnp.float32),
            pltpu.VMEM((ROWS_PER_W // 8, 8 * HID), jnp.float32),
            pltpu.SemaphoreType.DMA,
        ] + [pltpu.SemaphoreType.DMA] * NBUF,
        compiler_params=pltpu.CompilerParams(use_tc_tiling_on_sc=False),
    )
    def body(xa_hbm, xb_hbm, p_hbm, out_hbm, idx_a, idx_b, bufs, out_v,
             semi, *sems):
        wid = lax.axis_index("s") * NC + lax.axis_index("c")
        pltpu.async_copy(
            xa_hbm.at[pl.ds(ROWS_PER_W * wid, ROWS_PER_W), pl.ds(0, CHUNK_A)],
            idx_a, semi)
        pltpu.async_copy(
            xb_hbm.at[pl.ds(ROWS_PER_W * wid, ROWS_PER_W), pl.ds(0, CHUNK_B)],
            idx_b, semi).wait()
        pltpu.make_async_copy(
            xa_hbm.at[pl.ds(0, ROWS_PER_W), pl.ds(0, CHUNK_A)],
            idx_a, semi).wait()

        def fire_row(r, k):
            pltpu.async_copy(p_hbm.at[idx_a.at[r]],
                             bufs.at[k, pl.ds(0, CHUNK_A)], sems[k])
            pltpu.async_copy(p_hbm.at[idx_b.at[r]],
                             bufs.at[k, pl.ds(CHUNK_A, CHUNK_B)], sems[k])

        def wait_row(k):
            pltpu.make_async_copy(p_hbm.at[pl.ds(0, L)],
                                  bufs.at[k], sems[k]).wait()

        for k in range(NBUF):
            fire_row(k, k)

        def step(i, _):
            for k in range(STEP):
                kb = k % NBUF
                wait_row(kb)
                acc = _accum(bufs.at[kb])
                out_v[i, pl.ds(HID * k, HID)] = acc

                @pl.when(STEP * i + k + NBUF < ROWS_PER_W)
                def _():
                    fire_row(STEP * i + k + NBUF, kb)

            return 0

        lax.fori_loop(0, ROWS_PER_W // STEP, step, 0)
        pltpu.sync_copy(
            out_v,
            out_hbm.at[pl.ds(wid * (ROWS_PER_W // 8), ROWS_PER_W // 8)])

    return body(xa, xb, p)


def _head_body(s8_ref, yf_ref, m_ref, b2_ref, out_ref):
    h = jnp.maximum(s8_ref[...] * (1.0 / L), 0.0)
    lg = jnp.dot(h, m_ref[...], preferred_element_type=jnp.float32)
    b2 = b2_ref[...]
    l0 = lg[:, :8] + b2[0, 0]
    l1 = lg[:, 8:] + b2[0, 1]
    mx = jnp.maximum(l0, l1)
    lse = mx + jnp.log(jnp.exp(l0 - mx) + jnp.exp(l1 - mx))
    yf = yf_ref[...]
    picked = l0 + yf * (l1 - l0)
    out_ref[...] = (jnp.sum(lse - picked) * (1.0 / B)).reshape(1, 1)


def _head(s8, yf, m, b2row):
    return pl.pallas_call(
        _head_body,
        out_shape=jax.ShapeDtypeStruct((1, 1), jnp.float32),
    )(s8, yf, m, b2row)


def kernel(x_, y_, table, W1, b1, W2, b2):
    p8, xa, xb = _project(table, W1, b1.reshape(1, HID), x_.astype(jnp.int32))
    p = p8.reshape(VOCAB_PAD, HID)

    s8 = _gather_sum(xa, xb, p)

    eye8 = jnp.eye(8, dtype=jnp.float32)
    m = jnp.concatenate(
        [jnp.kron(eye8, W2[:, 0:1]), jnp.kron(eye8, W2[:, 1:2])], axis=1)
    yf = y_.astype(jnp.float32).reshape(B // 8, 8)

    out = _head(s8, yf, m, b2.reshape(1, CLASSES))
    return out[0, 0]
